# baseline (speedup 1.0000x reference)
import functools

import jax
import jax.numpy as jnp
from jax import lax
from jax.experimental import pallas as pl
from jax.experimental.pallas import tpu as pltpu
from jax.experimental.pallas import tpu_sc as plsc

N_NODES = 100000
D = 256
DW = D + 16
N_CLASSES = 32
N_GRAPHS = 128

SC_ROWS = 26000
CHUNK = 80
N_CHUNKS = SC_ROWS // CHUNK
NW = 32
VCHUNKS = 12

TC_BLK = 2000
TC_GRID = (N_NODES - SC_ROWS) // TC_BLK
TC_OFF = SC_ROWS // TC_BLK
TC_SPAN = 16
TC_OUT = N_GRAPHS + TC_SPAN


def _sc_segment_sums(embeddings, ids):
    mesh = plsc.VectorSubcoreMesh(core_axis_name="c", subcore_axis_name="s")

    @functools.partial(
        pl.kernel,
        mesh=mesh,
        out_type=jax.ShapeDtypeStruct((NW, N_GRAPHS, DW), jnp.float32),
        scratch_types=[
            pltpu.VMEM((CHUNK, D), jnp.float32),
            pltpu.VMEM((CHUNK, D), jnp.float32),
            pltpu.VMEM((CHUNK,), jnp.int32),
            pltpu.VMEM((CHUNK,), jnp.int32),
            pltpu.VMEM((N_GRAPHS, DW), jnp.float32),
            pltpu.SemaphoreType.DMA,
            pltpu.SemaphoreType.DMA,
            pltpu.SemaphoreType.DMA,
            pltpu.SemaphoreType.DMA,
        ],
    )
    def k(emb_hbm, ids_hbm, out_hbm,
          rows0, rows1, idx0, idx1, acc_v, sr0, sr1, si0, si1):
        c = lax.axis_index("c")
        s = lax.axis_index("s")
        w = c * 16 + s

        rows = (rows0, rows1)
        idxs = (idx0, idx1)
        srs = (sr0, sr1)
        sis = (si0, si1)

        zero16 = jnp.zeros((16,), jnp.float32)
        one16 = jnp.ones((16,), jnp.float32)
        sixteen16 = jnp.full((16,), 16.0, jnp.float32)
        chunk16 = jnp.full((16,), float(CHUNK), jnp.float32)

        def start(j, b):
            chunk = w + NW * j

            @pl.when(chunk < N_CHUNKS)
            def _():
                base = chunk * CHUNK
                pltpu.async_copy(emb_hbm.at[pl.ds(base, CHUNK)], rows[b],
                                 srs[b])
                pltpu.async_copy(ids_hbm.at[pl.ds(base, CHUNK)], idxs[b],
                                 sis[b])

        def wait(b):
            pltpu.make_async_copy(emb_hbm.at[pl.ds(0, CHUNK)], rows[b],
                                  srs[b]).wait()
            pltpu.make_async_copy(ids_hbm.at[pl.ds(0, CHUNK)], idxs[b],
                                  sis[b]).wait()

        start(0, 0)

        def zbody(i, carry):
            for d in range(DW // 16):
                acc_v[i, pl.ds(d * 16, 16)] = zero16
            return carry

        lax.fori_loop(0, N_GRAPHS, zbody, 0)

        def _tree(vals):
            while len(vals) > 1:
                nxt = [vals[2 * t] + vals[2 * t + 1]
                       for t in range(len(vals) // 2)]
                if len(vals) % 2:
                    nxt.append(vals[-1])
                vals = nxt
            return vals[0]

        def accumulate(rows_b, idx_b):
            seghead = idx_b[pl.ds(0, 16)]
            segtail = idx_b[pl.ds(CHUNK - 16, 16)]
            cfirst = seghead[0]
            clast = segtail[15]

            @pl.when(cfirst == clast)
            def _chunk_uniform():
                def dbody(dd, dcarry):
                    vals = [rows_b[r, pl.ds(dd * 16, 16)]
                            for r in range(CHUNK)]
                    plsc.addupdate(acc_v.at[cfirst, pl.ds(dd * 16, 16)],
                                   _tree(vals))
                    return dcarry

                lax.fori_loop(0, D // 16, dbody, 0)
                plsc.addupdate(acc_v.at[cfirst, pl.ds(D, 16)], chunk16)

            @pl.when(cfirst != clast)
            def _mixed():
                def grp_body(g, rcarry):
                    segs = idx_b[pl.ds(g * 16, 16)]
                    seg0 = segs[0]
                    seg15 = segs[15]

                    @pl.when(seg0 == seg15)
                    def _fast():
                        results = []
                        for d in range(D // 16):
                            vals = [rows_b[g * 16 + ii, pl.ds(d * 16, 16)]
                                    for ii in range(16)]
                            results.append(_tree(vals))
                        for d in range(D // 16):
                            plsc.addupdate(acc_v.at[seg0, pl.ds(d * 16, 16)],
                                           results[d])
                        plsc.addupdate(acc_v.at[seg0, pl.ds(D, 16)],
                                       sixteen16)

                    @pl.when(seg0 != seg15)
                    def _slow():
                        for ii in range(16):
                            seg = segs[ii]
                            ri = g * 16 + ii
                            for d in range(D // 16):
                                plsc.addupdate(
                                    acc_v.at[seg, pl.ds(d * 16, 16)],
                                    rows_b[ri, pl.ds(d * 16, 16)])
                            plsc.addupdate(acc_v.at[seg, pl.ds(D, 16)],
                                           one16)

                    return rcarry

                lax.fori_loop(0, CHUNK // 16, grp_body, 0)

        def pair_body(p, carry):
            for b in (0, 1):
                j = 2 * p + b
                chunk = w + NW * j
                start(j + 1, 1 - b)

                @pl.when(chunk < N_CHUNKS)
                def _(b=b):
                    wait(b)
                    accumulate(rows[b], idxs[b])

            return carry

        lax.fori_loop(0, VCHUNKS // 2, pair_body, 0)

        pltpu.sync_copy(acc_v, out_hbm.at[w])

    return k(embeddings, ids)


def _tc_partial(embeddings, ids3):
    def body(emb_ref, ids_ref, sums_ref, cnt_ref):
        i = pl.program_id(0)
        idsb = ids_ref[0, 0, :]
        lo = jnp.min(idsb)
        hi = jnp.max(idsb)
        base = pl.multiple_of((lo // 8) * 8, 8)

        @pl.when(i == 0)
        def _init():
            sums_ref[...] = jnp.zeros_like(sums_ref)
            cnt_ref[...] = jnp.zeros_like(cnt_ref)

        @pl.when(hi - base < TC_SPAN)
        def _narrow():
            gids = base + lax.broadcasted_iota(jnp.int32, (TC_SPAN, TC_BLK),
                                               0)
            oh = (idsb[None, :] == gids).astype(jnp.float32)
            part = jnp.dot(oh, emb_ref[...],
                           preferred_element_type=jnp.float32,
                           precision=lax.Precision.HIGHEST)
            sums_ref[pl.ds(base, TC_SPAN), :] += part
            cnt_ref[pl.ds(base, TC_SPAN), :] += jnp.sum(oh, axis=1,
                                                        keepdims=True)

        @pl.when(hi - base >= TC_SPAN)
        def _wide():
            gids = lax.broadcasted_iota(jnp.int32, (N_GRAPHS, TC_BLK), 0)
            oh = (idsb[None, :] == gids).astype(jnp.float32)
            part = jnp.dot(oh, emb_ref[...],
                           preferred_element_type=jnp.float32,
                           precision=lax.Precision.HIGHEST)
            sums_ref[pl.ds(0, N_GRAPHS), :] += part
            cnt_ref[pl.ds(0, N_GRAPHS), :] += jnp.sum(oh, axis=1,
                                                      keepdims=True)

    return pl.pallas_call(
        body,
        grid=(TC_GRID,),
        in_specs=[
            pl.BlockSpec((TC_BLK, D), lambda i: (i + TC_OFF, 0)),
            pl.BlockSpec((1, 1, TC_BLK), lambda i: (i + TC_OFF, 0, 0)),
        ],
        out_specs=[
            pl.BlockSpec((TC_OUT, D), lambda i: (0, 0)),
            pl.BlockSpec((TC_OUT, 1), lambda i: (0, 0)),
        ],
        out_shape=[
            jax.ShapeDtypeStruct((TC_OUT, D), jnp.float32),
            jax.ShapeDtypeStruct((TC_OUT, 1), jnp.float32),
        ],
    )(embeddings, ids3)


def _tc_finalize(sc_part, tc_sums, tc_cnt, w_t, bias):
    def body(ps, ts, tcnt, wt, bb, out):
        tot = jnp.sum(ps[...], axis=0)
        sums = tot[:, :D] + ts[:N_GRAPHS, :]
        cnt = jnp.maximum(tot[:, D:D + 1] + tcnt[:N_GRAPHS, :], 1.0)
        out[...] = (jnp.dot(sums, wt[...], preferred_element_type=jnp.float32)
                    / cnt + bb[...])

    return pl.pallas_call(
        body,
        out_shape=jax.ShapeDtypeStruct((N_GRAPHS, N_CLASSES), jnp.float32),
    )(sc_part, tc_sums, tc_cnt, w_t, bias)


def kernel(embeddings, batch, W, b):
    ids = batch.astype(jnp.int32)
    ids3 = ids.reshape(N_NODES // TC_BLK, 1, TC_BLK)
    sc_part = _sc_segment_sums(embeddings, ids)
    tc_sums, tc_cnt = _tc_partial(embeddings, ids3)
    return _tc_finalize(sc_part, tc_sums, tc_cnt, W.T,
                        b.reshape(1, N_CLASSES))


# speedup vs baseline: 1.1743x; 1.1743x over previous
import functools

import jax
import jax.numpy as jnp
from jax import lax
from jax.experimental import pallas as pl
from jax.experimental.pallas import tpu as pltpu
from jax.experimental.pallas import tpu_sc as plsc

N_NODES = 100000
D = 256
DW = D + 16
N_CLASSES = 32
N_GRAPHS = 128

SC_ROWS = 26000
CHUNK = 80
N_CHUNKS = SC_ROWS // CHUNK
NW = 32
VCHUNKS = 12

TC_BLK = 2000
TC_GRID = (N_NODES - SC_ROWS) // TC_BLK
TC_OFF = SC_ROWS // TC_BLK
TC_SPAN = 16
TC_OUT = N_GRAPHS + TC_SPAN


def _sc_segment_sums(embeddings, ids):
    mesh = plsc.VectorSubcoreMesh(core_axis_name="c", subcore_axis_name="s")

    @functools.partial(
        pl.kernel,
        mesh=mesh,
        out_type=jax.ShapeDtypeStruct((NW, N_GRAPHS, DW), jnp.float32),
        scratch_types=[
            pltpu.VMEM((CHUNK, D), jnp.float32),
            pltpu.VMEM((CHUNK, D), jnp.float32),
            pltpu.VMEM((CHUNK,), jnp.int32),
            pltpu.VMEM((CHUNK,), jnp.int32),
            pltpu.VMEM((N_GRAPHS, DW), jnp.float32),
            pltpu.SemaphoreType.DMA,
            pltpu.SemaphoreType.DMA,
            pltpu.SemaphoreType.DMA,
            pltpu.SemaphoreType.DMA,
        ],
    )
    def k(emb_hbm, ids_hbm, out_hbm,
          rows0, rows1, idx0, idx1, acc_v, sr0, sr1, si0, si1):
        c = lax.axis_index("c")
        s = lax.axis_index("s")
        w = c * 16 + s

        rows = (rows0, rows1)
        idxs = (idx0, idx1)
        srs = (sr0, sr1)
        sis = (si0, si1)

        zero16 = jnp.zeros((16,), jnp.float32)
        one16 = jnp.ones((16,), jnp.float32)
        sixteen16 = jnp.full((16,), 16.0, jnp.float32)
        chunk16 = jnp.full((16,), float(CHUNK), jnp.float32)

        def start(j, b):
            chunk = w + NW * j

            @pl.when(chunk < N_CHUNKS)
            def _():
                base = chunk * CHUNK
                pltpu.async_copy(emb_hbm.at[pl.ds(base, CHUNK)], rows[b],
                                 srs[b])
                pltpu.async_copy(ids_hbm.at[pl.ds(base, CHUNK)], idxs[b],
                                 sis[b])

        def wait(b):
            pltpu.make_async_copy(emb_hbm.at[pl.ds(0, CHUNK)], rows[b],
                                  srs[b]).wait()
            pltpu.make_async_copy(ids_hbm.at[pl.ds(0, CHUNK)], idxs[b],
                                  sis[b]).wait()

        start(0, 0)

        def zbody(i, carry):
            for d in range(DW // 16):
                acc_v[i, pl.ds(d * 16, 16)] = zero16
            return carry

        lax.fori_loop(0, N_GRAPHS, zbody, 0)

        def _tree(vals):
            while len(vals) > 1:
                nxt = [vals[2 * t] + vals[2 * t + 1]
                       for t in range(len(vals) // 2)]
                if len(vals) % 2:
                    nxt.append(vals[-1])
                vals = nxt
            return vals[0]

        def accumulate(rows_b, idx_b):
            seghead = idx_b[pl.ds(0, 16)]
            segtail = idx_b[pl.ds(CHUNK - 16, 16)]
            cfirst = seghead[0]
            clast = segtail[15]

            @pl.when(cfirst == clast)
            def _chunk_uniform():
                def dbody(dd, dcarry):
                    vals = [rows_b[r, pl.ds(dd * 16, 16)]
                            for r in range(CHUNK)]
                    plsc.addupdate(acc_v.at[cfirst, pl.ds(dd * 16, 16)],
                                   _tree(vals))
                    return dcarry

                lax.fori_loop(0, D // 16, dbody, 0)
                plsc.addupdate(acc_v.at[cfirst, pl.ds(D, 16)], chunk16)

            @pl.when(cfirst != clast)
            def _mixed():
                def grp_body(g, rcarry):
                    segs = idx_b[pl.ds(g * 16, 16)]
                    seg0 = segs[0]
                    seg15 = segs[15]

                    @pl.when(seg0 == seg15)
                    def _fast():
                        results = []
                        for d in range(D // 16):
                            vals = [rows_b[g * 16 + ii, pl.ds(d * 16, 16)]
                                    for ii in range(16)]
                            results.append(_tree(vals))
                        for d in range(D // 16):
                            plsc.addupdate(acc_v.at[seg0, pl.ds(d * 16, 16)],
                                           results[d])
                        plsc.addupdate(acc_v.at[seg0, pl.ds(D, 16)],
                                       sixteen16)

                    @pl.when(seg0 != seg15)
                    def _slow():
                        for ii in range(16):
                            seg = segs[ii]
                            ri = g * 16 + ii
                            for d in range(D // 16):
                                plsc.addupdate(
                                    acc_v.at[seg, pl.ds(d * 16, 16)],
                                    rows_b[ri, pl.ds(d * 16, 16)])
                            plsc.addupdate(acc_v.at[seg, pl.ds(D, 16)],
                                           one16)

                    return rcarry

                lax.fori_loop(0, CHUNK // 16, grp_body, 0)

        def pair_body(p, carry):
            for b in (0, 1):
                j = 2 * p + b
                chunk = w + NW * j
                start(j + 1, 1 - b)

                @pl.when(chunk < N_CHUNKS)
                def _(b=b):
                    wait(b)
                    accumulate(rows[b], idxs[b])

            return carry

        lax.fori_loop(0, VCHUNKS // 2, pair_body, 0)

        pltpu.sync_copy(acc_v, out_hbm.at[w])

    return k(embeddings, ids)


def _tc_partial(embeddings, ids3):
    def body(emb_ref, ids_ref, sums_ref, cnt_ref):
        i = pl.program_id(0)
        idsb = ids_ref[0, 0, :]
        lo = jnp.min(idsb)
        hi = jnp.max(idsb)
        base = pl.multiple_of((lo // 8) * 8, 8)

        @pl.when(i == 0)
        def _init():
            sums_ref[...] = jnp.zeros_like(sums_ref)
            cnt_ref[...] = jnp.zeros_like(cnt_ref)

        @pl.when(hi - base < TC_SPAN)
        def _narrow():
            gids = base + lax.broadcasted_iota(jnp.int32, (TC_SPAN, TC_BLK),
                                               0)
            eq = idsb[None, :] == gids
            oh16 = eq.astype(jnp.bfloat16)
            part = jnp.dot(oh16, emb_ref[...].astype(jnp.bfloat16),
                           preferred_element_type=jnp.float32)
            sums_ref[pl.ds(base, TC_SPAN), :] += part
            cnt_ref[pl.ds(base, TC_SPAN), :] += jnp.sum(
                eq.astype(jnp.float32), axis=1, keepdims=True)

        @pl.when(hi - base >= TC_SPAN)
        def _wide():
            gids = lax.broadcasted_iota(jnp.int32, (N_GRAPHS, TC_BLK), 0)
            oh = (idsb[None, :] == gids).astype(jnp.float32)
            part = jnp.dot(oh, emb_ref[...],
                           preferred_element_type=jnp.float32,
                           precision=lax.Precision.HIGHEST)
            sums_ref[pl.ds(0, N_GRAPHS), :] += part
            cnt_ref[pl.ds(0, N_GRAPHS), :] += jnp.sum(oh, axis=1,
                                                      keepdims=True)

    return pl.pallas_call(
        body,
        grid=(TC_GRID,),
        in_specs=[
            pl.BlockSpec((TC_BLK, D), lambda i: (i + TC_OFF, 0)),
            pl.BlockSpec((1, 1, TC_BLK), lambda i: (i + TC_OFF, 0, 0)),
        ],
        out_specs=[
            pl.BlockSpec((TC_OUT, D), lambda i: (0, 0)),
            pl.BlockSpec((TC_OUT, 1), lambda i: (0, 0)),
        ],
        out_shape=[
            jax.ShapeDtypeStruct((TC_OUT, D), jnp.float32),
            jax.ShapeDtypeStruct((TC_OUT, 1), jnp.float32),
        ],
    )(embeddings, ids3)


def _tc_finalize(sc_part, tc_sums, tc_cnt, w_t, bias):
    def body(ps, ts, tcnt, wt, bb, out):
        tot = jnp.sum(ps[...], axis=0)
        sums = tot[:, :D] + ts[:N_GRAPHS, :]
        cnt = jnp.maximum(tot[:, D:D + 1] + tcnt[:N_GRAPHS, :], 1.0)
        out[...] = (jnp.dot(sums, wt[...], preferred_element_type=jnp.float32)
                    / cnt + bb[...])

    return pl.pallas_call(
        body,
        out_shape=jax.ShapeDtypeStruct((N_GRAPHS, N_CLASSES), jnp.float32),
    )(sc_part, tc_sums, tc_cnt, w_t, bias)


def kernel(embeddings, batch, W, b):
    ids = batch.astype(jnp.int32)
    ids3 = ids.reshape(N_NODES // TC_BLK, 1, TC_BLK)
    sc_part = _sc_segment_sums(embeddings, ids)
    tc_sums, tc_cnt = _tc_partial(embeddings, ids3)
    return _tc_finalize(sc_part, tc_sums, tc_cnt, W.T,
                        b.reshape(1, N_CLASSES))
